# baseline (speedup 1.0000x reference)
import functools

import jax
import jax.numpy as jnp
from jax import lax
from jax.experimental import pallas as pl
from jax.experimental.pallas import tpu as pltpu
from jax.experimental.pallas import tpu_sc as plsc

N = 10000
K = 32
D = 128
H = 128

_BLOCK = 240
_FBLOCK = 80
N_SC = 2560
N_TC = N - N_SC

_NC = 2
_NS = 16
_NW = _NC * _NS
_ROWS_PER_W = N_SC // _NW
_CHUNK = 8
_NCHUNK = _ROWS_PER_W // _CHUNK
_LANES = 16


def _sc_reduce_chunk(buf, acc):
    for j in range(D // _LANES):
        sl = pl.ds(j * _LANES, _LANES)
        vecs = [buf[r * K, sl] for r in range(_CHUNK)]
        for k in range(1, K):
            for r in range(_CHUNK):
                vecs[r] = vecs[r] + buf[r * K + k, sl]
        for r in range(_CHUNK):
            acc[r, sl] = vecs[r]


def _sc_sum_body(nbr_hbm, out_hbm, buf0, buf1, acc, sem0, sem1, osem):
    wid = lax.axis_index("s") * _NC + lax.axis_index("c")
    row0 = N_TC + wid * _ROWS_PER_W
    out0 = wid * _ROWS_PER_W
    npair = _NCHUNK // 2

    def src(g):
        return nbr_hbm.at[pl.ds((row0 + g * _CHUNK) * K, _CHUNK * K)]

    def dst(g):
        return out_hbm.at[pl.ds(out0 + g * _CHUNK, _CHUNK)]

    pltpu.async_copy(src(0), buf0, sem0)
    pltpu.async_copy(src(1), buf1, sem1)

    def pair_body(p, carry):
        g = 2 * p
        pltpu.make_async_copy(src(0), buf0, sem0).wait()

        @pl.when(p < npair - 1)
        def _():
            pltpu.async_copy(src(g + 2), buf0, sem0)

        _sc_reduce_chunk(buf0, acc)
        pltpu.sync_copy(acc, dst(g))

        pltpu.make_async_copy(src(0), buf1, sem1).wait()

        @pl.when(p < npair - 1)
        def _():
            pltpu.async_copy(src(g + 3), buf1, sem1)

        _sc_reduce_chunk(buf1, acc)
        pltpu.sync_copy(acc, dst(g + 1))
        return carry

    lax.fori_loop(0, npair, pair_body, 0)


def _sc_neighbor_sums(nbr_flat):
    kfn = functools.partial(
        pl.kernel,
        mesh=plsc.VectorSubcoreMesh(core_axis_name="c", subcore_axis_name="s"),
        out_type=jax.ShapeDtypeStruct((N_SC, D), jnp.float32),
        scratch_types=[
            pltpu.VMEM((_CHUNK * K, D), jnp.float32),
            pltpu.VMEM((_CHUNK * K, D), jnp.float32),
            pltpu.VMEM((_CHUNK, D), jnp.float32),
            pltpu.SemaphoreType.DMA,
            pltpu.SemaphoreType.DMA,
            pltpu.SemaphoreType.DMA,
        ],
    )(_sc_sum_body)
    return kfn(nbr_flat)


def _tc_main_kernel(src_ref, nbr_ref, raw_ref, w_ref, b_ref, out_ref):
    aggr = jnp.sum(nbr_ref[...], axis=1) * (1.0 / K)
    neighbor_hidden = jnp.dot(aggr, w_ref[...],
                              preferred_element_type=jnp.float32)
    self_hidden = jnp.dot(src_ref[...], b_ref[...],
                          preferred_element_type=jnp.float32)
    hidden = neighbor_hidden + self_hidden
    out_ref[:, :H] = jnp.maximum(hidden, 0.0)
    out_ref[:, H:] = jnp.maximum(raw_ref[...], 0.0)


def _tc_finish_kernel(dummy_ref, sums_ref, src_ref, raw_ref, w_ref, b_ref,
                      out_ref):
    aggr = sums_ref[...] * (1.0 / K)
    neighbor_hidden = jnp.dot(aggr, w_ref[...],
                              preferred_element_type=jnp.float32)
    self_hidden = jnp.dot(src_ref[...], b_ref[...],
                          preferred_element_type=jnp.float32)
    hidden = neighbor_hidden + self_hidden
    out_ref[:, :H] = jnp.maximum(hidden, 0.0)
    out_ref[:, H:] = jnp.maximum(raw_ref[...], 0.0)


def kernel(src_node_features, neighbor_node_features, raw_data, W, b):
    nbr_flat = neighbor_node_features.reshape(N * K, D)

    sc_sums = _sc_neighbor_sums(nbr_flat)

    n_blocks_tc = N_TC // _BLOCK
    out_main = pl.pallas_call(
        _tc_main_kernel,
        grid=(n_blocks_tc,),
        in_specs=[
            pl.BlockSpec((_BLOCK, D), lambda i: (i, 0)),
            pl.BlockSpec((_BLOCK, K, D), lambda i: (i, 0, 0)),
            pl.BlockSpec((_BLOCK, D), lambda i: (i, 0)),
            pl.BlockSpec((D, H), lambda i: (0, 0)),
            pl.BlockSpec((D, H), lambda i: (0, 0)),
        ],
        out_specs=pl.BlockSpec((_BLOCK, 2 * H), lambda i: (i, 0)),
        out_shape=jax.ShapeDtypeStruct((N, 2 * H), jnp.float32),
    )(src_node_features, neighbor_node_features, raw_data, W, b)

    n_blocks_sc = N_SC // _FBLOCK
    off = N_TC // _FBLOCK
    out = pl.pallas_call(
        _tc_finish_kernel,
        grid=(n_blocks_sc,),
        in_specs=[
            pl.BlockSpec((_FBLOCK, 2 * H), lambda i: (i + off, 0)),
            pl.BlockSpec((_FBLOCK, D), lambda i: (i, 0)),
            pl.BlockSpec((_FBLOCK, D), lambda i: (i + off, 0)),
            pl.BlockSpec((_FBLOCK, D), lambda i: (i + off, 0)),
            pl.BlockSpec((D, H), lambda i: (0, 0)),
            pl.BlockSpec((D, H), lambda i: (0, 0)),
        ],
        out_specs=pl.BlockSpec((_FBLOCK, 2 * H), lambda i: (i + off, 0)),
        out_shape=jax.ShapeDtypeStruct((N, 2 * H), jnp.float32),
        input_output_aliases={0: 0},
    )(out_main, sc_sums, src_node_features, raw_data, W, b)
    return out


# speedup vs baseline: 2.4020x; 2.4020x over previous
import jax
import jax.numpy as jnp
from jax.experimental import pallas as pl
from jax.experimental.pallas import tpu as pltpu

_BLOCK = 512


def _sage_kernel(src_ref, nbr_ref, raw_ref, w_ref, b_ref, out_ref):
    k = nbr_ref.shape[1]
    aggr = jnp.sum(nbr_ref[...], axis=1) * (1.0 / k)
    neighbor_hidden = jnp.dot(aggr, w_ref[...],
                              preferred_element_type=jnp.float32)
    self_hidden = jnp.dot(src_ref[...], b_ref[...],
                          preferred_element_type=jnp.float32)
    hidden = neighbor_hidden + self_hidden
    h = hidden.shape[1]
    out_ref[:, :h] = jnp.maximum(hidden, 0.0)
    out_ref[:, h:] = jnp.maximum(raw_ref[...], 0.0)


def kernel(src_node_features, neighbor_node_features, raw_data, W, b):
    n, k, d = neighbor_node_features.shape
    h = W.shape[1]
    blk = _BLOCK
    grid = (pl.cdiv(n, blk),)
    return pl.pallas_call(
        _sage_kernel,
        grid=grid,
        in_specs=[
            pl.BlockSpec((blk, d), lambda i: (i, 0)),
            pl.BlockSpec((blk, k, d), lambda i: (i, 0, 0)),
            pl.BlockSpec((blk, d), lambda i: (i, 0)),
            pl.BlockSpec((d, h), lambda i: (0, 0)),
            pl.BlockSpec((d, h), lambda i: (0, 0)),
        ],
        out_specs=pl.BlockSpec((blk, 2 * h), lambda i: (i, 0)),
        out_shape=jax.ShapeDtypeStruct((n, 2 * h), jnp.float32),
    )(src_node_features, neighbor_node_features, raw_data, W, b)
